# baseline (speedup 1.0000x reference)
import functools

import jax
import jax.numpy as jnp
from jax import lax
from jax.experimental import pallas as pl
from jax.experimental.pallas import tpu as pltpu
from jax.experimental.pallas import tpu_sc as plsc

_NUM_EMBEDDINGS = 1024
_EMBEDDING_DIM = 64
_BETA = 0.25
_ROWS = 8 * 1024
_BLOCK_ROWS = 4096
_GRID = _ROWS // _BLOCK_ROWS


def _vq_block(x_ref, e_ref, idx_ref, loss_ref):
    x = x_ref[...]
    e = e_ref[...]
    scores = jax.lax.dot_general(
        x, e, (((1,), (0,)), ((), ())), preferred_element_type=jnp.float32)
    rowsq = jnp.sum(x ** 2, axis=1, keepdims=True)
    esq = jnp.sum(e ** 2, axis=0, keepdims=True)
    d = rowsq - 2.0 * scores + esq
    idx = jnp.argmin(d, axis=1)
    idx_ref[...] = idx.reshape(1, 1, _BLOCK_ROWS)
    part = jnp.sum(jnp.min(d, axis=1)).reshape(1, 1)

    @pl.when(pl.program_id(0) == 0)
    def _init():
        loss_ref[...] = jnp.zeros((1, 1), jnp.float32)

    loss_ref[...] += part


def _tc_stage(flat, embeddings):
    return pl.pallas_call(
        _vq_block,
        grid=(_GRID,),
        in_specs=[
            pl.BlockSpec((_BLOCK_ROWS, _EMBEDDING_DIM), lambda i: (i, 0)),
            pl.BlockSpec((_EMBEDDING_DIM, _NUM_EMBEDDINGS), lambda i: (0, 0)),
        ],
        out_specs=[
            pl.BlockSpec((1, 1, _BLOCK_ROWS), lambda i: (i, 0, 0)),
            pl.BlockSpec((1, 1), lambda i: (0, 0)),
        ],
        out_shape=[
            jax.ShapeDtypeStruct((_GRID, 1, _BLOCK_ROWS), jnp.int32),
            jax.ShapeDtypeStruct((1, 1), jnp.float32),
        ],
    )(flat, embeddings)


def _sc_gather(table2, idx_half):
    info = plsc.get_sparse_core_info()
    nw = info.num_cores * info.num_subcores
    b_per_w = _ROWS // nw
    mesh = plsc.VectorSubcoreMesh(core_axis_name="c", subcore_axis_name="s")

    @functools.partial(
        pl.kernel, mesh=mesh,
        out_type=jax.ShapeDtypeStruct((_ROWS, 2 * _EMBEDDING_DIM), jnp.float32),
        scratch_types=[
            pltpu.VMEM((b_per_w,), jnp.int32),
            pltpu.VMEM((b_per_w, 2 * _EMBEDDING_DIM), jnp.float32),
            pltpu.SemaphoreType.DMA,
        ],
    )
    def gather_kernel(table_hbm, idx_hbm, out_hbm, idx_v, rows_v, sem):
        wid = lax.axis_index("s") * info.num_cores + lax.axis_index("c")
        base = wid * b_per_w
        pltpu.sync_copy(idx_hbm.at[pl.ds(base, b_per_w)], idx_v)
        pltpu.async_copy(table_hbm.at[idx_v], rows_v, sem).wait()
        pltpu.sync_copy(rows_v, out_hbm.at[pl.ds(base, b_per_w)])

    return gather_kernel(table2, idx_half)


@functools.partial(jax.jit, static_argnames=())
def kernel(inputs, embeddings):
    input_shape = inputs.shape
    flat = inputs.reshape(_ROWS, _EMBEDDING_DIM)
    idx3, loss_sum = _tc_stage(flat, embeddings)
    encoding_indices = idx3.reshape(_ROWS)
    table2 = embeddings.T.reshape(_NUM_EMBEDDINGS // 2, 2 * _EMBEDDING_DIM)
    q128 = _sc_gather(table2, encoding_indices >> 1)
    q = jnp.where((encoding_indices & 1)[:, None] == 1,
                  q128[:, _EMBEDDING_DIM:], q128[:, :_EMBEDDING_DIM])
    quantized = inputs + (q.reshape(input_shape) - inputs)
    commitment_loss = _BETA * (loss_sum[0, 0] / (_ROWS * _EMBEDDING_DIM))
    return (quantized, commitment_loss, encoding_indices)


# speedup vs baseline: 2.3014x; 2.3014x over previous
import functools

import jax
import jax.numpy as jnp
from jax.experimental import pallas as pl

_NUM_EMBEDDINGS = 1024
_EMBEDDING_DIM = 64
_BETA = 0.25
_ROWS = 8 * 1024
_BLOCK_ROWS = 4096
_GRID = _ROWS // _BLOCK_ROWS


def _vq_block(x_ref, e_ref, q_ref, idx_ref, loss_ref):
    x = x_ref[...]
    e = e_ref[...]
    scores = jax.lax.dot_general(
        x, e, (((1,), (0,)), ((), ())), preferred_element_type=jnp.float32)
    rowsq = jnp.sum(x ** 2, axis=1, keepdims=True)
    esq = jnp.sum(e ** 2, axis=0, keepdims=True)
    d = rowsq - 2.0 * scores + esq
    idx = jnp.argmin(d, axis=1)
    onehot = (jax.lax.broadcasted_iota(jnp.int32, (_BLOCK_ROWS, _NUM_EMBEDDINGS), 1)
              == idx[:, None]).astype(jnp.float32)
    q = jax.lax.dot_general(
        onehot, e, (((1,), (1,)), ((), ())), preferred_element_type=jnp.float32)
    q_ref[...] = x + (q - x)
    idx_ref[...] = idx.reshape(1, 1, _BLOCK_ROWS)
    diff = q - x
    part = jnp.sum(diff * diff).reshape(1, 1)

    @pl.when(pl.program_id(0) == 0)
    def _init():
        loss_ref[...] = jnp.zeros((1, 1), jnp.float32)

    loss_ref[...] += part

    @pl.when(pl.program_id(0) == _GRID - 1)
    def _finish():
        loss_ref[...] = loss_ref[...] * (_BETA / (_ROWS * _EMBEDDING_DIM))


@functools.partial(jax.jit, static_argnames=())
def kernel(inputs, embeddings):
    input_shape = inputs.shape
    flat = inputs.reshape(_ROWS, _EMBEDDING_DIM)
    q_flat, idx3, loss_sum = pl.pallas_call(
        _vq_block,
        grid=(_GRID,),
        in_specs=[
            pl.BlockSpec((_BLOCK_ROWS, _EMBEDDING_DIM), lambda i: (i, 0)),
            pl.BlockSpec((_EMBEDDING_DIM, _NUM_EMBEDDINGS), lambda i: (0, 0)),
        ],
        out_specs=[
            pl.BlockSpec((_BLOCK_ROWS, _EMBEDDING_DIM), lambda i: (i, 0)),
            pl.BlockSpec((1, 1, _BLOCK_ROWS), lambda i: (i, 0, 0)),
            pl.BlockSpec((1, 1), lambda i: (0, 0)),
        ],
        out_shape=[
            jax.ShapeDtypeStruct((_ROWS, _EMBEDDING_DIM), jnp.float32),
            jax.ShapeDtypeStruct((_GRID, 1, _BLOCK_ROWS), jnp.int32),
            jax.ShapeDtypeStruct((1, 1), jnp.float32),
        ],
    )(flat, embeddings)
    quantized = q_flat.reshape(input_shape)
    commitment_loss = loss_sum.reshape(())
    encoding_indices = idx3.reshape(_ROWS)
    return (quantized, commitment_loss, encoding_indices)
